# baseline (speedup 1.0000x reference)
import functools

import jax
import jax.numpy as jnp
from jax import lax
from jax.experimental import pallas as pl
from jax.experimental.pallas import tpu as pltpu
from jax.experimental.pallas import tpu_sc as plsc

HIDDEN = 1024
TROWS = 40
MASK_ID = 32
ZERO_ROW = 33
LN_EPS = 1e-12

_INFO = plsc.get_sparse_core_info()
NC, NS = _INFO.num_cores, _INFO.num_subcores
NW = NC * NS
CHUNK = 16
NBUF = 4


def _make_prep(b_per_w):
    def prep(w_ref, g_ref, b_ref, ids_ref, mask_ref, t_ref, idx_ref):
        w = w_ref[...]
        mu = jnp.mean(w, axis=1, keepdims=True)
        var = jnp.mean((w - mu) ** 2, axis=1, keepdims=True)
        normed = (w - mu) * lax.rsqrt(var + LN_EPS) * g_ref[...] + b_ref[...]
        r = lax.broadcasted_iota(jnp.int32, (TROWS, HIDDEN), 0)
        t = jnp.where(r == MASK_ID, b_ref[...], normed)
        t_ref[...] = jnp.where(r >= ZERO_ROW, 0.0, t)
        idx_ref[...] = jnp.where(mask_ref[...] != 0.0, ids_ref[...], ZERO_ROW)

    return prep


def _make_gather(total):
    b_per_w = total // NW
    nchunk = b_per_w // CHUNK
    mesh = plsc.VectorSubcoreMesh(core_axis_name="c", subcore_axis_name="s")

    @functools.partial(
        pl.kernel,
        mesh=mesh,
        out_type=jax.ShapeDtypeStruct((total * HIDDEN,), jnp.float32),
        scratch_types=[
            pltpu.SMEM((b_per_w,), jnp.int32),
            pltpu.VMEM((b_per_w,), jnp.int32),
            pltpu.VMEM_SHARED((TROWS * HIDDEN,), jnp.float32),
            pltpu.VMEM((CHUNK * HIDDEN,), jnp.float32),
            pltpu.VMEM((CHUNK * HIDDEN,), jnp.float32),
            pltpu.SemaphoreType.DMA,
            pltpu.SemaphoreType.DMA,
            pltpu.SemaphoreType.DMA,
        ],
    )
    def gather(t_hbm, idx_hbm, out_hbm, idx_s, idx_v, table_v, rows0, rows1,
               s0, s1, csem):
        wid = lax.axis_index("s") * NC + lax.axis_index("c")
        base = wid * b_per_w
        @pl.when(lax.axis_index("s") == 0)
        def _():
            pltpu.sync_copy(t_hbm, table_v)

        pltpu.sync_copy(idx_hbm.at[pl.ds(base, b_per_w)], idx_v)
        plsc.subcore_barrier()

        rows = (rows0, rows1)
        ssem = (s0, s1)

        def s_copy(k, b):
            return pltpu.make_async_copy(
                rows[b],
                out_hbm.at[pl.ds((base + k * CHUNK) * HIDDEN, CHUNK * HIDDEN)],
                ssem[b],
            )

        def construct(k, b):
            def grp_body(g, carry):
                v = idx_v[pl.ds(k * CHUNK + g * 16, 16)]
                for t in range(16):
                    pltpu.async_copy(
                        table_v.at[pl.ds(v[t] * HIDDEN, HIDDEN)],
                        rows[b].at[pl.ds((g * 16 + t) * HIDDEN, HIDDEN)],
                        csem,
                    )
                for t in range(16):
                    pltpu.make_async_copy(
                        table_v.at[pl.ds(v[t] * HIDDEN, HIDDEN)],
                        rows[b].at[pl.ds((g * 16 + t) * HIDDEN, HIDDEN)],
                        csem,
                    ).wait()
                return carry

            lax.fori_loop(0, CHUNK // 16, grp_body, 0)

        def body(i, _):
            for b in range(2):
                k = i * 2 + b

                @pl.when(k >= 2)
                def _():
                    s_copy(k - 2, b).wait()

                construct(k, b)
                s_copy(k, b).start()
            return 0

        lax.fori_loop(0, nchunk // 2, body, 0)
        s_copy(nchunk - 2, 0).wait()
        s_copy(nchunk - 1, 1).wait()

    return gather


def kernel(input_ids, attention_mask, W, gamma, beta):
    B, S = input_ids.shape
    total = B * S
    b_per_w = total // NW
    ids32 = input_ids.astype(jnp.int32)
    w_pad = jnp.zeros((TROWS, HIDDEN), jnp.float32).at[: W.shape[0]].set(W)

    table, idx = pl.pallas_call(
        _make_prep(b_per_w),
        out_shape=(
            jax.ShapeDtypeStruct((TROWS, HIDDEN), jnp.float32),
            jax.ShapeDtypeStruct((B, S), jnp.int32),
        ),
    )(w_pad, gamma.reshape(1, HIDDEN), beta.reshape(1, HIDDEN), ids32,
      attention_mask)

    out = _make_gather(total)(table.reshape(TROWS * HIDDEN), idx.reshape(total))
    return out.reshape(B, S, HIDDEN)


# speedup vs baseline: 1.9898x; 1.9898x over previous
import functools

import jax
import jax.numpy as jnp
from jax import lax
from jax.experimental import pallas as pl
from jax.experimental.pallas import tpu as pltpu
from jax.experimental.pallas import tpu_sc as plsc

HIDDEN = 1024
TROWS = 40
MASK_ID = 32
ZERO_ROW = 33
LN_EPS = 1e-12

_INFO = plsc.get_sparse_core_info()
NC, NS = _INFO.num_cores, _INFO.num_subcores
NW = NC * NS
CHUNK = 16
NBUF = 4
NREP = 16


def _make_prep(b_per_w):
    def prep(w_ref, g_ref, b_ref, ids_ref, mask_ref, t_ref, idx_ref):
        w = w_ref[...]
        mu = jnp.mean(w, axis=1, keepdims=True)
        var = jnp.mean((w - mu) ** 2, axis=1, keepdims=True)
        normed = (w - mu) * lax.rsqrt(var + LN_EPS) * g_ref[...] + b_ref[...]
        r = lax.broadcasted_iota(jnp.int32, (TROWS, HIDDEN), 0)
        t = jnp.where(r == MASK_ID, b_ref[...], normed)
        t_ref[...] = jnp.where(r >= ZERO_ROW, 0.0, t)
        idx = jnp.where(mask_ref[...] != 0.0, ids_ref[...], ZERO_ROW)
        shape = idx.shape
        flat = (
            lax.broadcasted_iota(jnp.int32, shape, 0) * shape[1]
            + lax.broadcasted_iota(jnp.int32, shape, 1)
        )
        idx_ref[...] = idx + ((flat // b_per_w) // 2) * TROWS

    return prep


def _make_gather(total):
    b_per_w = total // NW
    nchunk = b_per_w // CHUNK
    mesh = plsc.VectorSubcoreMesh(core_axis_name="c", subcore_axis_name="s")

    @functools.partial(
        pl.kernel,
        mesh=mesh,
        out_type=jax.ShapeDtypeStruct((total, HIDDEN), jnp.float32),
        scratch_types=(
            [
                pltpu.VMEM((b_per_w,), jnp.int32),
                pltpu.VMEM_SHARED((NREP * TROWS, HIDDEN), jnp.float32),
            ]
            + [pltpu.VMEM((CHUNK, HIDDEN), jnp.float32) for _ in range(NBUF)]
            + [pltpu.SemaphoreType.DMA for _ in range(2 * NBUF + 1)]
        ),
    )
    def gather(t_hbm, idx_hbm, out_hbm, idx_v, table_sh, *bufs):
        rows = bufs[:NBUF]
        gsem = bufs[NBUF : 2 * NBUF]
        ssem = bufs[2 * NBUF : 3 * NBUF]
        csem = bufs[3 * NBUF]
        wid = lax.axis_index("s") * NC + lax.axis_index("c")
        base = wid * b_per_w
        @pl.when(lax.axis_index("s") == 0)
        def _():
            pltpu.sync_copy(t_hbm, table_sh)

        pltpu.sync_copy(idx_hbm.at[pl.ds(base, b_per_w)], idx_v)
        plsc.subcore_barrier()

        def g_copy(k, b):
            return pltpu.make_async_copy(
                t_hbm.at[idx_v.at[pl.ds(k * CHUNK, CHUNK)]], rows[b],
                gsem[b],
            )

        def s_copy(k, b):
            return pltpu.make_async_copy(
                rows[b],
                out_hbm.at[pl.ds(base + k * CHUNK, CHUNK)],
                ssem[b],
            )

        def construct(k, b):
            def grp_body(g, carry):
                v = idx_v[pl.ds(k * CHUNK + g * 16, 16)]
                for t in range(16):
                    pltpu.async_copy(
                        table_sh.at[v[t]],
                        rows[b].at[g * 16 + t],
                        csem,
                    )
                for t in range(16):
                    pltpu.make_async_copy(
                        table_sh.at[v[t]],
                        rows[b].at[g * 16 + t],
                        csem,
                    ).wait()
                return carry

            lax.fori_loop(0, CHUNK // 16, grp_body, 0)

        g_copy(0, 0).start()
        g_copy(2, 2).start()

        def body(i, _):
            for b in range(NBUF):
                k = i * NBUF + b
                if b % 2 == 0:
                    g_copy(k, b).wait()
                    s_copy(k, b).start()
                else:
                    @pl.when(k + 3 < nchunk)
                    def _():
                        @pl.when(k >= 1)
                        def _():
                            s_copy(k - 1, b - 1).wait()

                        g_copy(k + 3, b - 1).start()

                    @pl.when(k >= NBUF)
                    def _():
                        s_copy(k - NBUF, b).wait()

                    construct(k, b)
                    s_copy(k, b).start()
            return 0

        lax.fori_loop(0, nchunk // NBUF, body, 0)
        for j in range(NBUF):
            k = nchunk - NBUF + j
            s_copy(k, k % NBUF).wait()

    return gather


def kernel(input_ids, attention_mask, W, gamma, beta):
    B, S = input_ids.shape
    total = B * S
    b_per_w = total // NW
    ids32 = input_ids.astype(jnp.int32)
    w_pad = jnp.zeros((TROWS, HIDDEN), jnp.float32).at[: W.shape[0]].set(W)

    table, idx = pl.pallas_call(
        _make_prep(b_per_w),
        out_shape=(
            jax.ShapeDtypeStruct((TROWS, HIDDEN), jnp.float32),
            jax.ShapeDtypeStruct((B, S), jnp.int32),
        ),
    )(w_pad, gamma.reshape(1, HIDDEN), beta.reshape(1, HIDDEN), ids32,
      attention_mask)

    table_rep = jnp.tile(table, (NREP, 1))
    out = _make_gather(total)(table_rep, idx.reshape(total))
    return out.reshape(B, S, HIDDEN)
